# baseline (speedup 1.0000x reference)
import functools

import jax
import jax.numpy as jnp
from jax import lax
from jax.experimental import pallas as pl
from jax.experimental.pallas import tpu as pltpu
from jax.experimental.pallas import tpu_sc as plsc

N = 10000
E = 320000
D = 128
ATTN = 64

NC = 2
NS = 16
NW = NC * NS
EPT = E // NW
CH = 80
NCHT = EPT // CH
GRP = 25
NGRP = NCHT // GRP
ROW_A = 640
ROW_B = N - 15 * ROW_A
VEC_A = 1280
NVP = 8 * VEC_A
ZR = 80


def _sc_body(src_i, dst_i, src_a, dst_a, hs_hbm, gs_hbm, u_hbm, v_hbm,
             sint_out, cnt_out, sa_out, den_out,
             src_g, dst_g, ug, vg, exg, ones_g, zvec, rows0, rows1,
             rsem0, rsem1, usem0, usem1, vsem0, vsem1,
             srsem0, srsem1, svsem0, svsem1,
             sh_rows, sh_vec):
    c = lax.axis_index("c")
    s = lax.axis_index("s")
    wid = c * NS + s
    f32 = jnp.float32
    zero16 = jnp.zeros((16,), f32)
    rows = (rows0, rows1)
    rsem = (rsem0, rsem1)
    srsem = (srsem0, srsem1)

    @pl.loop(0, VEC_A // 16)
    def _(i):
        zvec[pl.ds(i * 16, 16)] = zero16

    one16 = jnp.ones((16,), f32)

    @pl.loop(0, GRP)
    def _(r):
        for j in range(CH // 16):
            ones_g[r, pl.ds(j * 16, 16)] = one16

    def zero_shared():
        @pl.loop(0, ZR)
        def _(r):
            for j in range(D // 16):
                rows0[r, pl.ds(j * 16, 16)] = zero16

        @pl.when(s < 15)
        def _():
            for b in range(ROW_A // ZR):
                pltpu.sync_copy(rows0, sh_rows.at[pl.ds(s * ROW_A + b * ZR, ZR)])

        @pl.when(s == 15)
        def _():
            for b in range(ROW_B // ZR):
                pltpu.sync_copy(rows0, sh_rows.at[pl.ds(15 * ROW_A + b * ZR, ZR)])

        @pl.when(s < 8)
        def _():
            pltpu.sync_copy(zvec, sh_vec.at[pl.ds(s * VEC_A, VEC_A)])

    def flush(out3, outv):
        @pl.when(s < 15)
        def _():
            pltpu.sync_copy(sh_rows.at[pl.ds(s * ROW_A, ROW_A)],
                            out3.at[c, pl.ds(s * ROW_A, ROW_A)])

        @pl.when(s == 15)
        def _():
            pltpu.sync_copy(sh_rows.at[pl.ds(15 * ROW_A, ROW_B)],
                            out3.at[c, pl.ds(15 * ROW_A, ROW_B)])

        @pl.when(s < 8)
        def _():
            pltpu.sync_copy(sh_vec.at[pl.ds(s * VEC_A, VEC_A)],
                            outv.at[c, pl.ds(s * VEC_A, VEC_A)])

    def gather_start(kk, b, table):
        pltpu.async_copy(table.at[src_g.at[kk]], rows[b], rsem[b])

    def gather_wait(kk, b, table):
        pltpu.make_async_copy(table.at[src_g.at[kk]], rows[b], rsem[b]).wait()

    def scatter_start(kk, b):
        pltpu.async_copy(rows[b], sh_rows.at[dst_g.at[kk]], srsem[b], add=True)

    def scatter_wait(kk, b):
        pltpu.make_async_copy(rows[b], sh_rows.at[dst_g.at[kk]], srsem[b]).wait()

    def compute_scale(kk, b):
        for j in range(CH // 16):
            sl = pl.ds(j * 16, 16)
            a = ug[kk, sl] + vg[kk, sl]
            a = jnp.maximum(a, a * 0.01)
            exg[kk, sl] = jnp.exp(a)

        @pl.loop(0, CH // 16)
        def _(g):
            r0 = g * 16
            exv = exg[kk, pl.ds(r0, 16)]
            for l in range(16):
                ev = jnp.broadcast_to(exv[l], (16,))
                for j in range(D // 16):
                    sl = pl.ds(j * 16, 16)
                    rows[b][r0 + l, sl] = rows[b][r0 + l, sl] * ev

    def chunk(kk, b, table, attn):
        gather_wait(kk, b, table)
        kkm1 = jnp.maximum(kk - 1, 0)

        @pl.when(kk >= 1)
        def _():
            scatter_wait(kkm1, 1 - b)

        @pl.when(kk + 1 < GRP)
        def _():
            gather_start(jnp.minimum(kk + 1, GRP - 1), 1 - b, table)

        if attn:
            compute_scale(kk, b)
        scatter_start(kk, b)
        vsrc = exg.at[kk] if attn else ones_g.at[kk]
        pltpu.async_copy(vsrc, sh_vec.at[dst_g.at[kk]], svsem0, add=True)

    def phase(table, si3, di3, attn, out3, outv):
        zero_shared()
        plsc.subcore_barrier()

        @pl.loop(0, NGRP)
        def _(g):
            pltpu.sync_copy(si3.at[wid, g], src_g)
            pltpu.sync_copy(di3.at[wid, g], dst_g)
            if attn:
                @pl.loop(0, GRP)
                def _(kk):
                    pltpu.async_copy(u_hbm.at[src_g.at[kk]], ug.at[kk], usem0)
                    pltpu.async_copy(v_hbm.at[dst_g.at[kk]], vg.at[kk], vsem0)

            gather_start(0, 0, table)

            if attn:
                @pl.loop(0, GRP)
                def _(kk):
                    pltpu.make_async_copy(u_hbm.at[src_g.at[kk]], ug.at[kk],
                                          usem0).wait()
                    pltpu.make_async_copy(v_hbm.at[dst_g.at[kk]], vg.at[kk],
                                          vsem0).wait()

            @pl.loop(0, GRP - 1, step=2)
            def _(k):
                chunk(k, 0, table, attn)
                chunk(k + 1, 1, table, attn)

            chunk(GRP - 1, 0, table, attn)
            scatter_wait(GRP - 1, 0)

            @pl.loop(0, GRP)
            def _(kk):
                vsrc = exg.at[kk] if attn else ones_g.at[kk]
                pltpu.make_async_copy(vsrc, sh_vec.at[dst_g.at[kk]],
                                      svsem0).wait()

        plsc.subcore_barrier()
        flush(out3, outv)

    phase(hs_hbm, src_i, dst_i, False, sint_out, cnt_out)
    phase(gs_hbm, src_a, dst_a, True, sa_out, den_out)


_sc_edges = pl.kernel(
    _sc_body,
    out_type=(
        jax.ShapeDtypeStruct((NC, N, D), jnp.float32),
        jax.ShapeDtypeStruct((NC, NVP), jnp.float32),
        jax.ShapeDtypeStruct((NC, N, D), jnp.float32),
        jax.ShapeDtypeStruct((NC, NVP), jnp.float32),
    ),
    mesh=plsc.VectorSubcoreMesh(core_axis_name="c", subcore_axis_name="s"),
    scratch_types=(
        [pltpu.VMEM((GRP, CH), jnp.int32)] * 2
        + [pltpu.VMEM((GRP, CH), jnp.float32)] * 4
        + [pltpu.VMEM((VEC_A,), jnp.float32)]
        + [pltpu.VMEM((CH, D), jnp.float32)] * 2
        + [pltpu.SemaphoreType.DMA] * 10
        + [pltpu.VMEM_SHARED((N, D), jnp.float32),
           pltpu.VMEM_SHARED((NVP,), jnp.float32)]
    ),
)


BN = 1000


def _prep_body(h_ref, ai_ref, aa_ref, wna_ref, w2_ref, hs_ref, gs_ref, uv_ref):
    hb = h_ref[...]
    hs_ref[...] = jnp.dot(hb, ai_ref[...], preferred_element_type=jnp.float32)
    gs_ref[...] = jnp.dot(hb, aa_ref[...], preferred_element_type=jnp.float32)
    z = jnp.dot(hb, wna_ref[...], preferred_element_type=jnp.float32)
    uv_ref[...] = jnp.dot(z, w2_ref[...], preferred_element_type=jnp.float32)


def _final_body(h_ref, sint_ref, cnt_ref, sa_ref, den_ref, wid_ref, wad_ref,
                ci_ref, ca_ref, wu1_ref, wu2_ref, bu_ref, out_ref):
    hb = h_ref[...]
    si = sint_ref[0] + sint_ref[1]
    cnt = cnt_ref[0] + cnt_ref[1]
    sa = sa_ref[0] + sa_ref[1]
    den = den_ref[0] + den_ref[1]
    pi = jnp.dot(hb, wid_ref[...], preferred_element_type=jnp.float32) + ci_ref[...]
    pa = jnp.dot(hb, wad_ref[...], preferred_element_type=jnp.float32) + ca_ref[...]
    msg_i = (si + cnt * pi) / jnp.maximum(cnt, 1.0)
    msg_a = (sa + den * pa) / jnp.maximum(den, 1e-9)
    o = (jnp.dot(msg_i, wu1_ref[...], preferred_element_type=jnp.float32)
         + jnp.dot(msg_a, wu2_ref[...], preferred_element_type=jnp.float32)
         + bu_ref[...])
    out_ref[...] = jnp.maximum(o, 0.0)


def _row_spec(width):
    return pl.BlockSpec((BN, width), lambda i: (i, 0))


def _full_spec(shape):
    nd = len(shape)
    return pl.BlockSpec(shape, lambda i, _n=nd: (0,) * _n)


@jax.jit
def kernel(h, edge_index_intra, edge_index_inter, W_msg_intra, b_msg_intra,
           W_msg_inter, b_msg_inter, ef_intra, ef_inter, W_node_attn, W_attn,
           W_update, b_update):
    f32 = jnp.float32
    h = h.astype(f32)

    ai = W_msg_intra[:, :D].T
    aa = W_msg_inter[:, :D].T
    wid_w = W_msg_intra[:, D:2 * D].T
    wad_w = W_msg_inter[:, D:2 * D].T
    ci = (W_msg_intra[:, 2 * D:] @ ef_intra + b_msg_intra)[None, :]
    ca = (W_msg_inter[:, 2 * D:] @ ef_inter + b_msg_inter)[None, :]
    wna_t = W_node_attn.T
    w2 = W_attn.reshape(2, ATTN).T
    wu1 = W_update[:, :D].T
    wu2 = W_update[:, D:].T
    bu = b_update[None, :]

    hs, gs, uv = pl.pallas_call(
        _prep_body,
        grid=(N // BN,),
        in_specs=[
            _row_spec(D),
            _full_spec((D, D)),
            _full_spec((D, D)),
            _full_spec((D, ATTN)),
            _full_spec((ATTN, 2)),
        ],
        out_specs=[_row_spec(D), _row_spec(D), _row_spec(2)],
        out_shape=[
            jax.ShapeDtypeStruct((N, D), f32),
            jax.ShapeDtypeStruct((N, D), f32),
            jax.ShapeDtypeStruct((N, 2), f32),
        ],
    )(h, ai, aa, wna_t, w2)

    u = uv[:, 0] + 0.0
    v = uv[:, 1] + 0.0

    src_i = edge_index_intra[0].astype(jnp.int32).reshape(NW, NGRP, GRP, CH)
    dst_i = edge_index_intra[1].astype(jnp.int32).reshape(NW, NGRP, GRP, CH)
    src_a = edge_index_inter[0].astype(jnp.int32).reshape(NW, NGRP, GRP, CH)
    dst_a = edge_index_inter[1].astype(jnp.int32).reshape(NW, NGRP, GRP, CH)

    sint, cnt, sa, den = _sc_edges(src_i, dst_i, src_a, dst_a, hs, gs, u, v)

    out = pl.pallas_call(
        _final_body,
        grid=(N // BN,),
        in_specs=[
            _row_spec(D),
            pl.BlockSpec((NC, BN, D), lambda i: (0, i, 0)),
            pl.BlockSpec((NC, BN, 1), lambda i: (0, i, 0)),
            pl.BlockSpec((NC, BN, D), lambda i: (0, i, 0)),
            pl.BlockSpec((NC, BN, 1), lambda i: (0, i, 0)),
            _full_spec((D, D)),
            _full_spec((D, D)),
            _full_spec((1, D)),
            _full_spec((1, D)),
            _full_spec((D, D)),
            _full_spec((D, D)),
            _full_spec((1, D)),
        ],
        out_specs=_row_spec(D),
        out_shape=jax.ShapeDtypeStruct((N, D), f32),
    )(h, sint, cnt[:, :N, None], sa, den[:, :N, None], wid_w, wad_w, ci, ca,
      wu1, wu2, bu)

    return out


# speedup vs baseline: 1.2844x; 1.2844x over previous
import functools

import jax
import jax.numpy as jnp
from jax import lax
from jax.experimental import pallas as pl
from jax.experimental.pallas import tpu as pltpu
from jax.experimental.pallas import tpu_sc as plsc

N = 10000
E = 320000
D = 128
ATTN = 64

NC = 2
NS = 16
NW = NC * NS
EPT = E // NW
CH = 80
NCHT = EPT // CH
GRP = 25
NGRP = NCHT // GRP
NB = 3
ROW_A = 640
ROW_B = N - 15 * ROW_A
VEC_A = 1280
NVP = 8 * VEC_A
ZR = 80


def _sc_body(src_i, dst_i, src_a, dst_a, hs_hbm, gs_hbm, u_hbm, v_hbm,
             sint_out, cnt_out, sa_out, den_out,
             src_g, dst_g, ub0, ub1, ub2, vb0, vb1, vb2, ex0, ex1, ex2,
             ones, zvec, rows0, rows1, rows2,
             rsem0, rsem1, rsem2, usem0, usem1, usem2,
             vsem0, vsem1, vsem2, srsem0, srsem1, srsem2,
             svsem0, svsem1, svsem2,
             sh_rows, sh_vec):
    c = lax.axis_index("c")
    s = lax.axis_index("s")
    wid = c * NS + s
    f32 = jnp.float32
    zero16 = jnp.zeros((16,), f32)
    rows = (rows0, rows1, rows2)
    ub = (ub0, ub1, ub2)
    vb = (vb0, vb1, vb2)
    ex = (ex0, ex1, ex2)
    rsem = (rsem0, rsem1, rsem2)
    usem = (usem0, usem1, usem2)
    vsem = (vsem0, vsem1, vsem2)
    srsem = (srsem0, srsem1, srsem2)
    svsem = (svsem0, svsem1, svsem2)

    @pl.loop(0, VEC_A // 16)
    def _(i):
        zvec[pl.ds(i * 16, 16)] = zero16

    for j in range(CH // 16):
        ones[pl.ds(j * 16, 16)] = jnp.ones((16,), f32)

    def zero_shared():
        @pl.loop(0, ZR)
        def _(r):
            for j in range(D // 16):
                rows0[r, pl.ds(j * 16, 16)] = zero16

        @pl.when(s < 15)
        def _():
            for b in range(ROW_A // ZR):
                pltpu.sync_copy(rows0, sh_rows.at[pl.ds(s * ROW_A + b * ZR, ZR)])

        @pl.when(s == 15)
        def _():
            for b in range(ROW_B // ZR):
                pltpu.sync_copy(rows0, sh_rows.at[pl.ds(15 * ROW_A + b * ZR, ZR)])

        @pl.when(s < 8)
        def _():
            pltpu.sync_copy(zvec, sh_vec.at[pl.ds(s * VEC_A, VEC_A)])

    def flush(out3, outv):
        @pl.when(s < 15)
        def _():
            pltpu.sync_copy(sh_rows.at[pl.ds(s * ROW_A, ROW_A)],
                            out3.at[c, pl.ds(s * ROW_A, ROW_A)])

        @pl.when(s == 15)
        def _():
            pltpu.sync_copy(sh_rows.at[pl.ds(15 * ROW_A, ROW_B)],
                            out3.at[c, pl.ds(15 * ROW_A, ROW_B)])

        @pl.when(s < 8)
        def _():
            pltpu.sync_copy(sh_vec.at[pl.ds(s * VEC_A, VEC_A)],
                            outv.at[c, pl.ds(s * VEC_A, VEC_A)])

    def gather_start(kk, b, table, attn):
        pltpu.async_copy(table.at[src_g.at[kk]], rows[b], rsem[b])
        if attn:
            pltpu.async_copy(u_hbm.at[src_g.at[kk]], ub[b], usem[b])
            pltpu.async_copy(v_hbm.at[dst_g.at[kk]], vb[b], vsem[b])

    def gather_wait(kk, b, table, attn):
        pltpu.make_async_copy(table.at[src_g.at[kk]], rows[b], rsem[b]).wait()
        if attn:
            pltpu.make_async_copy(u_hbm.at[src_g.at[kk]], ub[b], usem[b]).wait()
            pltpu.make_async_copy(v_hbm.at[dst_g.at[kk]], vb[b], vsem[b]).wait()

    def scatter_start(kk, b, attn):
        pltpu.async_copy(rows[b], sh_rows.at[dst_g.at[kk]], srsem[b], add=True)
        vsrc = ex[b] if attn else ones
        pltpu.async_copy(vsrc, sh_vec.at[dst_g.at[kk]], svsem[b], add=True)

    def scatter_wait(kk, b, attn):
        pltpu.make_async_copy(rows[b], sh_rows.at[dst_g.at[kk]], srsem[b]).wait()
        vsrc = ex[b] if attn else ones
        pltpu.make_async_copy(vsrc, sh_vec.at[dst_g.at[kk]], svsem[b]).wait()

    def compute_scale(kk, b):
        for j in range(CH // 16):
            sl = pl.ds(j * 16, 16)
            a = ub[b][sl] + vb[b][sl]
            a = jnp.maximum(a, a * 0.01)
            ex[b][sl] = jnp.exp(a)

        @pl.loop(0, CH // 16)
        def _(g):
            r0 = g * 16
            exv = ex[b][pl.ds(r0, 16)]
            for l in range(16):
                ev = jnp.broadcast_to(exv[l], (16,))
                for j in range(D // 16):
                    sl = pl.ds(j * 16, 16)
                    rows[b][r0 + l, sl] = rows[b][r0 + l, sl] * ev

    def chunk(kk, b, table, attn):
        gather_wait(kk, b, table, attn)
        bp = (b + NB - 1) % NB
        kkm1 = jnp.maximum(kk - 1, 0)

        @pl.when(kk >= 1)
        def _():
            scatter_wait(kkm1, bp, attn)

        @pl.when(kk + NB - 1 < GRP)
        def _():
            gather_start(jnp.minimum(kk + NB - 1, GRP - 1), bp, table, attn)

        if attn:
            compute_scale(kk, b)
        scatter_start(kk, b, attn)

    def phase(table, si3, di3, attn, out3, outv):
        zero_shared()
        plsc.subcore_barrier()

        @pl.loop(0, NGRP)
        def _(g):
            pltpu.sync_copy(si3.at[wid, g], src_g)
            pltpu.sync_copy(di3.at[wid, g], dst_g)
            for i in range(NB - 1):
                gather_start(i, i, table, attn)

            @pl.loop(0, GRP - 1, step=NB)
            def _(k):
                for i in range(NB):
                    chunk(k + i, i, table, attn)

            chunk(GRP - 1, (GRP - 1) % NB, table, attn)
            scatter_wait(GRP - 1, (GRP - 1) % NB, attn)

        plsc.subcore_barrier()
        flush(out3, outv)

    phase(hs_hbm, src_i, dst_i, False, sint_out, cnt_out)
    phase(gs_hbm, src_a, dst_a, True, sa_out, den_out)


_sc_edges = pl.kernel(
    _sc_body,
    out_type=(
        jax.ShapeDtypeStruct((NC, N, D), jnp.float32),
        jax.ShapeDtypeStruct((NC, NVP), jnp.float32),
        jax.ShapeDtypeStruct((NC, N, D), jnp.float32),
        jax.ShapeDtypeStruct((NC, NVP), jnp.float32),
    ),
    mesh=plsc.VectorSubcoreMesh(core_axis_name="c", subcore_axis_name="s"),
    scratch_types=(
        [pltpu.VMEM((GRP, CH), jnp.int32)] * 2
        + [pltpu.VMEM((CH,), jnp.float32)] * 10
        + [pltpu.VMEM((VEC_A,), jnp.float32)]
        + [pltpu.VMEM((CH, D), jnp.float32)] * 3
        + [pltpu.SemaphoreType.DMA] * 15
        + [pltpu.VMEM_SHARED((N, D), jnp.float32),
           pltpu.VMEM_SHARED((NVP,), jnp.float32)]
    ),
)


BN = 1000


def _prep_body(h_ref, ai_ref, aa_ref, wna_ref, w2_ref, hs_ref, gs_ref, uv_ref):
    hb = h_ref[...]
    hs_ref[...] = jnp.dot(hb, ai_ref[...], preferred_element_type=jnp.float32)
    gs_ref[...] = jnp.dot(hb, aa_ref[...], preferred_element_type=jnp.float32)
    z = jnp.dot(hb, wna_ref[...], preferred_element_type=jnp.float32)
    uv_ref[...] = jnp.dot(z, w2_ref[...], preferred_element_type=jnp.float32)


def _final_body(h_ref, sint_ref, cnt_ref, sa_ref, den_ref, wid_ref, wad_ref,
                ci_ref, ca_ref, wu1_ref, wu2_ref, bu_ref, out_ref):
    hb = h_ref[...]
    si = sint_ref[0] + sint_ref[1]
    cnt = cnt_ref[0] + cnt_ref[1]
    sa = sa_ref[0] + sa_ref[1]
    den = den_ref[0] + den_ref[1]
    pi = jnp.dot(hb, wid_ref[...], preferred_element_type=jnp.float32) + ci_ref[...]
    pa = jnp.dot(hb, wad_ref[...], preferred_element_type=jnp.float32) + ca_ref[...]
    msg_i = (si + cnt * pi) / jnp.maximum(cnt, 1.0)
    msg_a = (sa + den * pa) / jnp.maximum(den, 1e-9)
    o = (jnp.dot(msg_i, wu1_ref[...], preferred_element_type=jnp.float32)
         + jnp.dot(msg_a, wu2_ref[...], preferred_element_type=jnp.float32)
         + bu_ref[...])
    out_ref[...] = jnp.maximum(o, 0.0)


def _row_spec(width):
    return pl.BlockSpec((BN, width), lambda i: (i, 0))


def _full_spec(shape):
    nd = len(shape)
    return pl.BlockSpec(shape, lambda i, _n=nd: (0,) * _n)


@jax.jit
def kernel(h, edge_index_intra, edge_index_inter, W_msg_intra, b_msg_intra,
           W_msg_inter, b_msg_inter, ef_intra, ef_inter, W_node_attn, W_attn,
           W_update, b_update):
    f32 = jnp.float32
    h = h.astype(f32)

    ai = W_msg_intra[:, :D].T
    aa = W_msg_inter[:, :D].T
    wid_w = W_msg_intra[:, D:2 * D].T
    wad_w = W_msg_inter[:, D:2 * D].T
    ci = (W_msg_intra[:, 2 * D:] @ ef_intra + b_msg_intra)[None, :]
    ca = (W_msg_inter[:, 2 * D:] @ ef_inter + b_msg_inter)[None, :]
    wna_t = W_node_attn.T
    w2 = W_attn.reshape(2, ATTN).T
    wu1 = W_update[:, :D].T
    wu2 = W_update[:, D:].T
    bu = b_update[None, :]

    hs, gs, uv = pl.pallas_call(
        _prep_body,
        grid=(N // BN,),
        in_specs=[
            _row_spec(D),
            _full_spec((D, D)),
            _full_spec((D, D)),
            _full_spec((D, ATTN)),
            _full_spec((ATTN, 2)),
        ],
        out_specs=[_row_spec(D), _row_spec(D), _row_spec(2)],
        out_shape=[
            jax.ShapeDtypeStruct((N, D), f32),
            jax.ShapeDtypeStruct((N, D), f32),
            jax.ShapeDtypeStruct((N, 2), f32),
        ],
    )(h, ai, aa, wna_t, w2)

    u = uv[:, 0] + 0.0
    v = uv[:, 1] + 0.0

    src_i = edge_index_intra[0].astype(jnp.int32).reshape(NW, NGRP, GRP, CH)
    dst_i = edge_index_intra[1].astype(jnp.int32).reshape(NW, NGRP, GRP, CH)
    src_a = edge_index_inter[0].astype(jnp.int32).reshape(NW, NGRP, GRP, CH)
    dst_a = edge_index_inter[1].astype(jnp.int32).reshape(NW, NGRP, GRP, CH)

    sint, cnt, sa, den = _sc_edges(src_i, dst_i, src_a, dst_a, hs, gs, u, v)

    out = pl.pallas_call(
        _final_body,
        grid=(N // BN,),
        in_specs=[
            _row_spec(D),
            pl.BlockSpec((NC, BN, D), lambda i: (0, i, 0)),
            pl.BlockSpec((NC, BN, 1), lambda i: (0, i, 0)),
            pl.BlockSpec((NC, BN, D), lambda i: (0, i, 0)),
            pl.BlockSpec((NC, BN, 1), lambda i: (0, i, 0)),
            _full_spec((D, D)),
            _full_spec((D, D)),
            _full_spec((1, D)),
            _full_spec((1, D)),
            _full_spec((D, D)),
            _full_spec((D, D)),
            _full_spec((1, D)),
        ],
        out_specs=_row_spec(D),
        out_shape=jax.ShapeDtypeStruct((N, D), f32),
    )(h, sint, cnt[:, :N, None], sa, den[:, :N, None], wid_w, wad_w, ci, ca,
      wu1, wu2, bu)

    return out


# speedup vs baseline: 1.2865x; 1.0016x over previous
import functools

import jax
import jax.numpy as jnp
from jax import lax
from jax.experimental import pallas as pl
from jax.experimental.pallas import tpu as pltpu
from jax.experimental.pallas import tpu_sc as plsc

N = 10000
E = 320000
D = 128
ATTN = 64

NC = 2
NS = 16
NW = NC * NS
EPT = E // NW
CH = 80
NCHT = EPT // CH
GRP = 25
NGRP = NCHT // GRP
NB = 3
ROW_A = 640
ROW_B = N - 15 * ROW_A
VEC_A = 1280
NVP = 8 * VEC_A
ZR = 80


def _sc_body(src_i, dst_i, src_a, dst_a, hs_hbm, gs_hbm, u_hbm, v_hbm,
             sint_out, cnt_out, sa_out, den_out,
             src_g, dst_g, ub0, ub1, ub2, vb0, vb1, vb2, ex0, ex1, ex2,
             ones, zvec, rows0, rows1, rows2,
             rsem0, rsem1, rsem2, usem0, usem1, usem2,
             vsem0, vsem1, vsem2, srsem0, srsem1, srsem2,
             svsem0, svsem1, svsem2,
             sh_rows, sh_vec):
    c = lax.axis_index("c")
    s = lax.axis_index("s")
    wid = c * NS + s
    f32 = jnp.float32
    zero16 = jnp.zeros((16,), f32)
    rows = (rows0, rows1, rows2)
    ub = (ub0, ub1, ub2)
    vb = (vb0, vb1, vb2)
    ex = (ex0, ex1, ex2)
    rsem = (rsem0, rsem1, rsem2)
    usem = (usem0, usem1, usem2)
    vsem = (vsem0, vsem1, vsem2)
    srsem = (srsem0, srsem1, srsem2)
    svsem = (svsem0, svsem1, svsem2)

    @pl.loop(0, VEC_A // 16)
    def _(i):
        zvec[pl.ds(i * 16, 16)] = zero16

    for j in range(CH // 16):
        ones[pl.ds(j * 16, 16)] = jnp.ones((16,), f32)

    def zero_shared():
        @pl.loop(0, ZR)
        def _(r):
            for j in range(D // 16):
                rows0[r, pl.ds(j * 16, 16)] = zero16

        @pl.when(s < 15)
        def _():
            for b in range(ROW_A // ZR):
                pltpu.async_copy(rows0,
                                 sh_rows.at[pl.ds(s * ROW_A + b * ZR, ZR)],
                                 rsem0)
            for b in range(ROW_A // ZR):
                pltpu.make_async_copy(
                    rows0, sh_rows.at[pl.ds(s * ROW_A + b * ZR, ZR)],
                    rsem0).wait()

        @pl.when(s == 15)
        def _():
            for b in range(ROW_B // ZR):
                pltpu.async_copy(rows0,
                                 sh_rows.at[pl.ds(15 * ROW_A + b * ZR, ZR)],
                                 rsem0)
            for b in range(ROW_B // ZR):
                pltpu.make_async_copy(
                    rows0, sh_rows.at[pl.ds(15 * ROW_A + b * ZR, ZR)],
                    rsem0).wait()

        @pl.when(s < 8)
        def _():
            pltpu.sync_copy(zvec, sh_vec.at[pl.ds(s * VEC_A, VEC_A)])

    def flush(out3, outv):
        @pl.when(s < 15)
        def _():
            pltpu.sync_copy(sh_rows.at[pl.ds(s * ROW_A, ROW_A)],
                            out3.at[c, pl.ds(s * ROW_A, ROW_A)])

        @pl.when(s == 15)
        def _():
            pltpu.sync_copy(sh_rows.at[pl.ds(15 * ROW_A, ROW_B)],
                            out3.at[c, pl.ds(15 * ROW_A, ROW_B)])

        @pl.when(s < 8)
        def _():
            pltpu.sync_copy(sh_vec.at[pl.ds(s * VEC_A, VEC_A)],
                            outv.at[c, pl.ds(s * VEC_A, VEC_A)])

    def gather_start(kk, b, table, attn):
        pltpu.async_copy(table.at[src_g.at[kk]], rows[b], rsem[b])
        if attn:
            pltpu.async_copy(u_hbm.at[src_g.at[kk]], ub[b], usem[b])
            pltpu.async_copy(v_hbm.at[dst_g.at[kk]], vb[b], vsem[b])

    def gather_wait(kk, b, table, attn):
        pltpu.make_async_copy(table.at[src_g.at[kk]], rows[b], rsem[b]).wait()
        if attn:
            pltpu.make_async_copy(u_hbm.at[src_g.at[kk]], ub[b], usem[b]).wait()
            pltpu.make_async_copy(v_hbm.at[dst_g.at[kk]], vb[b], vsem[b]).wait()

    def scatter_start(kk, b, attn):
        pltpu.async_copy(rows[b], sh_rows.at[dst_g.at[kk]], srsem[b], add=True)
        vsrc = ex[b] if attn else ones
        pltpu.async_copy(vsrc, sh_vec.at[dst_g.at[kk]], svsem[b], add=True)

    def scatter_wait(kk, b, attn):
        pltpu.make_async_copy(rows[b], sh_rows.at[dst_g.at[kk]], srsem[b]).wait()
        vsrc = ex[b] if attn else ones
        pltpu.make_async_copy(vsrc, sh_vec.at[dst_g.at[kk]], svsem[b]).wait()

    def compute_scale(kk, b):
        for j in range(CH // 16):
            sl = pl.ds(j * 16, 16)
            a = ub[b][sl] + vb[b][sl]
            a = jnp.maximum(a, a * 0.01)
            ex[b][sl] = jnp.exp(a)

        @pl.loop(0, CH // 16)
        def _(g):
            r0 = g * 16
            exv = ex[b][pl.ds(r0, 16)]
            for l in range(16):
                ev = jnp.broadcast_to(exv[l], (16,))
                for j in range(D // 16):
                    sl = pl.ds(j * 16, 16)
                    rows[b][r0 + l, sl] = rows[b][r0 + l, sl] * ev

    def chunk(kk, b, table, attn):
        gather_wait(kk, b, table, attn)
        bp = (b + NB - 1) % NB
        kkm1 = jnp.maximum(kk - 1, 0)

        @pl.when(kk >= 1)
        def _():
            scatter_wait(kkm1, bp, attn)

        @pl.when(kk + NB - 1 < GRP)
        def _():
            gather_start(jnp.minimum(kk + NB - 1, GRP - 1), bp, table, attn)

        if attn:
            compute_scale(kk, b)
        scatter_start(kk, b, attn)

    def phase(table, si3, di3, attn, out3, outv):
        zero_shared()
        plsc.subcore_barrier()

        @pl.loop(0, NGRP)
        def _(g):
            pltpu.sync_copy(si3.at[wid, g], src_g)
            pltpu.sync_copy(di3.at[wid, g], dst_g)
            for i in range(NB - 1):
                gather_start(i, i, table, attn)

            @pl.loop(0, GRP - 1, step=NB)
            def _(k):
                for i in range(NB):
                    chunk(k + i, i, table, attn)

            chunk(GRP - 1, (GRP - 1) % NB, table, attn)
            scatter_wait(GRP - 1, (GRP - 1) % NB, attn)

        plsc.subcore_barrier()
        flush(out3, outv)

    phase(hs_hbm, src_i, dst_i, False, sint_out, cnt_out)
    phase(gs_hbm, src_a, dst_a, True, sa_out, den_out)


_sc_edges = pl.kernel(
    _sc_body,
    out_type=(
        jax.ShapeDtypeStruct((NC, N, D), jnp.float32),
        jax.ShapeDtypeStruct((NC, NVP), jnp.float32),
        jax.ShapeDtypeStruct((NC, N, D), jnp.float32),
        jax.ShapeDtypeStruct((NC, NVP), jnp.float32),
    ),
    mesh=plsc.VectorSubcoreMesh(core_axis_name="c", subcore_axis_name="s"),
    scratch_types=(
        [pltpu.VMEM((GRP, CH), jnp.int32)] * 2
        + [pltpu.VMEM((CH,), jnp.float32)] * 10
        + [pltpu.VMEM((VEC_A,), jnp.float32)]
        + [pltpu.VMEM((CH, D), jnp.float32)] * 3
        + [pltpu.SemaphoreType.DMA] * 15
        + [pltpu.VMEM_SHARED((N, D), jnp.float32),
           pltpu.VMEM_SHARED((NVP,), jnp.float32)]
    ),
)


BN = 1000


def _prep_body(h_ref, ai_ref, aa_ref, wna_ref, w2_ref, hs_ref, gs_ref, uv_ref):
    hb = h_ref[...]
    hs_ref[...] = jnp.dot(hb, ai_ref[...], preferred_element_type=jnp.float32)
    gs_ref[...] = jnp.dot(hb, aa_ref[...], preferred_element_type=jnp.float32)
    z = jnp.dot(hb, wna_ref[...], preferred_element_type=jnp.float32)
    uv_ref[...] = jnp.dot(z, w2_ref[...], preferred_element_type=jnp.float32)


def _final_body(h_ref, sint_ref, cnt_ref, sa_ref, den_ref, wid_ref, wad_ref,
                ci_ref, ca_ref, wu1_ref, wu2_ref, bu_ref, out_ref):
    hb = h_ref[...]
    si = sint_ref[0] + sint_ref[1]
    cnt = cnt_ref[0] + cnt_ref[1]
    sa = sa_ref[0] + sa_ref[1]
    den = den_ref[0] + den_ref[1]
    pi = jnp.dot(hb, wid_ref[...], preferred_element_type=jnp.float32) + ci_ref[...]
    pa = jnp.dot(hb, wad_ref[...], preferred_element_type=jnp.float32) + ca_ref[...]
    msg_i = (si + cnt * pi) / jnp.maximum(cnt, 1.0)
    msg_a = (sa + den * pa) / jnp.maximum(den, 1e-9)
    o = (jnp.dot(msg_i, wu1_ref[...], preferred_element_type=jnp.float32)
         + jnp.dot(msg_a, wu2_ref[...], preferred_element_type=jnp.float32)
         + bu_ref[...])
    out_ref[...] = jnp.maximum(o, 0.0)


def _row_spec(width):
    return pl.BlockSpec((BN, width), lambda i: (i, 0))


def _full_spec(shape):
    nd = len(shape)
    return pl.BlockSpec(shape, lambda i, _n=nd: (0,) * _n)


@jax.jit
def kernel(h, edge_index_intra, edge_index_inter, W_msg_intra, b_msg_intra,
           W_msg_inter, b_msg_inter, ef_intra, ef_inter, W_node_attn, W_attn,
           W_update, b_update):
    f32 = jnp.float32
    h = h.astype(f32)

    ai = W_msg_intra[:, :D].T
    aa = W_msg_inter[:, :D].T
    wid_w = W_msg_intra[:, D:2 * D].T
    wad_w = W_msg_inter[:, D:2 * D].T
    ci = (W_msg_intra[:, 2 * D:] @ ef_intra + b_msg_intra)[None, :]
    ca = (W_msg_inter[:, 2 * D:] @ ef_inter + b_msg_inter)[None, :]
    wna_t = W_node_attn.T
    w2 = W_attn.reshape(2, ATTN).T
    wu1 = W_update[:, :D].T
    wu2 = W_update[:, D:].T
    bu = b_update[None, :]

    hs, gs, uv = pl.pallas_call(
        _prep_body,
        grid=(N // BN,),
        in_specs=[
            _row_spec(D),
            _full_spec((D, D)),
            _full_spec((D, D)),
            _full_spec((D, ATTN)),
            _full_spec((ATTN, 2)),
        ],
        out_specs=[_row_spec(D), _row_spec(D), _row_spec(2)],
        out_shape=[
            jax.ShapeDtypeStruct((N, D), f32),
            jax.ShapeDtypeStruct((N, D), f32),
            jax.ShapeDtypeStruct((N, 2), f32),
        ],
    )(h, ai, aa, wna_t, w2)

    u = uv[:, 0] + 0.0
    v = uv[:, 1] + 0.0

    src_i = edge_index_intra[0].astype(jnp.int32).reshape(NW, NGRP, GRP, CH)
    dst_i = edge_index_intra[1].astype(jnp.int32).reshape(NW, NGRP, GRP, CH)
    src_a = edge_index_inter[0].astype(jnp.int32).reshape(NW, NGRP, GRP, CH)
    dst_a = edge_index_inter[1].astype(jnp.int32).reshape(NW, NGRP, GRP, CH)

    sint, cnt, sa, den = _sc_edges(src_i, dst_i, src_a, dst_a, hs, gs, u, v)

    out = pl.pallas_call(
        _final_body,
        grid=(N // BN,),
        in_specs=[
            _row_spec(D),
            pl.BlockSpec((NC, BN, D), lambda i: (0, i, 0)),
            pl.BlockSpec((NC, BN, 1), lambda i: (0, i, 0)),
            pl.BlockSpec((NC, BN, D), lambda i: (0, i, 0)),
            pl.BlockSpec((NC, BN, 1), lambda i: (0, i, 0)),
            _full_spec((D, D)),
            _full_spec((D, D)),
            _full_spec((1, D)),
            _full_spec((1, D)),
            _full_spec((D, D)),
            _full_spec((D, D)),
            _full_spec((1, D)),
        ],
        out_specs=_row_spec(D),
        out_shape=jax.ShapeDtypeStruct((N, D), f32),
    )(h, sint, cnt[:, :N, None], sa, den[:, :N, None], wid_w, wad_w, ci, ca,
      wu1, wu2, bu)

    return out


# speedup vs baseline: 1.3028x; 1.0126x over previous
import functools

import jax
import jax.numpy as jnp
from jax import lax
from jax.experimental import pallas as pl
from jax.experimental.pallas import tpu as pltpu
from jax.experimental.pallas import tpu_sc as plsc

N = 10000
E = 320000
D = 128
ATTN = 64

NC = 2
NS = 16
NW = NC * NS
EPT = E // NW
CH = 80
NCHT = EPT // CH
GRP = 25
NGRP = NCHT // GRP
NB = 3
ROW_A = 640
ROW_B = N - 15 * ROW_A
VEC_A = 1280
NVP = 8 * VEC_A
ZR = 80


def _sc_body(src_i, dst_i, src_a, dst_a, hs_hbm, gs_hbm, u_hbm, v_hbm,
             sint_out, cnt_out, sa_out, den_out,
             src_g, dst_g, ub0, ub1, ub2, vb0, vb1, vb2, ex0, ex1, ex2,
             ones, zvec, rows0, rows1, rows2,
             rsem0, rsem1, rsem2, usem0, usem1, usem2,
             vsem0, vsem1, vsem2, srsem0, srsem1, srsem2,
             svsem0, svsem1, svsem2,
             sh_rows, sh_vec):
    c = lax.axis_index("c")
    s = lax.axis_index("s")
    wid = c * NS + s
    f32 = jnp.float32
    zero16 = jnp.zeros((16,), f32)
    rows = (rows0, rows1, rows2)
    ub = (ub0, ub1, ub2)
    vb = (vb0, vb1, vb2)
    ex = (ex0, ex1, ex2)
    rsem = (rsem0, rsem1, rsem2)
    usem = (usem0, usem1, usem2)
    vsem = (vsem0, vsem1, vsem2)
    srsem = (srsem0, srsem1, srsem2)
    svsem = (svsem0, svsem1, svsem2)

    @pl.loop(0, VEC_A // 16)
    def _(i):
        zvec[pl.ds(i * 16, 16)] = zero16

    for j in range(CH // 16):
        ones[pl.ds(j * 16, 16)] = jnp.ones((16,), f32)

    def zero_shared():
        @pl.loop(0, ZR)
        def _(r):
            for j in range(D // 16):
                rows0[r, pl.ds(j * 16, 16)] = zero16

        @pl.when(s < 15)
        def _():
            for b in range(ROW_A // ZR):
                pltpu.async_copy(rows0,
                                 sh_rows.at[pl.ds(s * ROW_A + b * ZR, ZR)],
                                 rsem0)
            for b in range(ROW_A // ZR):
                pltpu.make_async_copy(
                    rows0, sh_rows.at[pl.ds(s * ROW_A + b * ZR, ZR)],
                    rsem0).wait()

        @pl.when(s == 15)
        def _():
            for b in range(ROW_B // ZR):
                pltpu.async_copy(rows0,
                                 sh_rows.at[pl.ds(15 * ROW_A + b * ZR, ZR)],
                                 rsem0)
            for b in range(ROW_B // ZR):
                pltpu.make_async_copy(
                    rows0, sh_rows.at[pl.ds(15 * ROW_A + b * ZR, ZR)],
                    rsem0).wait()

        @pl.when(s < 8)
        def _():
            pltpu.sync_copy(zvec, sh_vec.at[pl.ds(s * VEC_A, VEC_A)])

    def flush(out3, outv):
        @pl.when(s < 15)
        def _():
            pltpu.sync_copy(sh_rows.at[pl.ds(s * ROW_A, ROW_A)],
                            out3.at[c, pl.ds(s * ROW_A, ROW_A)])

        @pl.when(s == 15)
        def _():
            pltpu.sync_copy(sh_rows.at[pl.ds(15 * ROW_A, ROW_B)],
                            out3.at[c, pl.ds(15 * ROW_A, ROW_B)])

        @pl.when(s < 8)
        def _():
            pltpu.sync_copy(sh_vec.at[pl.ds(s * VEC_A, VEC_A)],
                            outv.at[c, pl.ds(s * VEC_A, VEC_A)])

    def gather_start(kk, b, table, attn):
        pltpu.async_copy(table.at[src_g.at[kk]], rows[b], rsem[b])
        if attn:
            pltpu.async_copy(u_hbm.at[src_g.at[kk]], ub[b], usem[b])
            pltpu.async_copy(v_hbm.at[dst_g.at[kk]], vb[b], vsem[b])

    def gather_wait(kk, b, table, attn):
        pltpu.make_async_copy(table.at[src_g.at[kk]], rows[b], rsem[b]).wait()
        if attn:
            pltpu.make_async_copy(u_hbm.at[src_g.at[kk]], ub[b], usem[b]).wait()
            pltpu.make_async_copy(v_hbm.at[dst_g.at[kk]], vb[b], vsem[b]).wait()

    def scatter_start(kk, b, attn):
        pltpu.async_copy(rows[b], sh_rows.at[dst_g.at[kk]], srsem[b], add=True)
        vsrc = ex[b] if attn else ones
        pltpu.async_copy(vsrc, sh_vec.at[dst_g.at[kk]], svsem[b], add=True)

    def scatter_wait(kk, b, attn):
        pltpu.make_async_copy(rows[b], sh_rows.at[dst_g.at[kk]], srsem[b]).wait()
        vsrc = ex[b] if attn else ones
        pltpu.make_async_copy(vsrc, sh_vec.at[dst_g.at[kk]], svsem[b]).wait()

    def compute_scale(kk, b):
        for j in range(CH // 16):
            sl = pl.ds(j * 16, 16)
            a = ub[b][sl] + vb[b][sl]
            a = jnp.maximum(a, a * 0.01)
            ex[b][sl] = jnp.exp(a)

        @pl.loop(0, CH // 16)
        def _(g):
            r0 = g * 16
            exv = ex[b][pl.ds(r0, 16)]
            for l in range(16):
                ev = jnp.broadcast_to(exv[l], (16,))
                for j in range(D // 16):
                    sl = pl.ds(j * 16, 16)
                    rows[b][r0 + l, sl] = rows[b][r0 + l, sl] * ev

    def chunk(kk, b, table, attn):
        gather_wait(kk, b, table, attn)
        bp = (b + NB - 1) % NB
        kkm1 = jnp.maximum(kk - 1, 0)

        @pl.when(kk >= 1)
        def _():
            scatter_wait(kkm1, bp, attn)

        @pl.when(kk + NB - 1 < GRP)
        def _():
            gather_start(jnp.minimum(kk + NB - 1, GRP - 1), bp, table, attn)

        if attn:
            compute_scale(kk, b)
        scatter_start(kk, b, attn)

    def phase(table, si3, di3, attn, out3, outv):
        zero_shared()
        plsc.subcore_barrier()

        @pl.loop(0, NGRP)
        def _(g):
            pltpu.sync_copy(si3.at[wid, g], src_g)
            pltpu.sync_copy(di3.at[wid, g], dst_g)
            for i in range(NB - 1):
                gather_start(i, i, table, attn)

            @pl.loop(0, GRP - 1, step=NB)
            def _(k):
                for i in range(NB):
                    chunk(k + i, i, table, attn)

            chunk(GRP - 1, (GRP - 1) % NB, table, attn)
            scatter_wait(GRP - 1, (GRP - 1) % NB, attn)

        plsc.subcore_barrier()
        flush(out3, outv)

    phase(hs_hbm, src_i, dst_i, False, sint_out, cnt_out)
    phase(gs_hbm, src_a, dst_a, True, sa_out, den_out)


_sc_edges = pl.kernel(
    _sc_body,
    out_type=(
        jax.ShapeDtypeStruct((NC, N, D), jnp.float32),
        jax.ShapeDtypeStruct((NC, NVP), jnp.float32),
        jax.ShapeDtypeStruct((NC, N, D), jnp.float32),
        jax.ShapeDtypeStruct((NC, NVP), jnp.float32),
    ),
    mesh=plsc.VectorSubcoreMesh(core_axis_name="c", subcore_axis_name="s"),
    scratch_types=(
        [pltpu.VMEM((GRP, CH), jnp.int32)] * 2
        + [pltpu.VMEM((CH,), jnp.float32)] * 10
        + [pltpu.VMEM((VEC_A,), jnp.float32)]
        + [pltpu.VMEM((CH, D), jnp.float32)] * 3
        + [pltpu.SemaphoreType.DMA] * 15
        + [pltpu.VMEM_SHARED((N, D), jnp.float32),
           pltpu.VMEM_SHARED((NVP,), jnp.float32)]
    ),
)


BN = 2000


def _prep_body(h_ref, ai_ref, aa_ref, wna_ref, w2_ref, hs_ref, gs_ref, uv_ref):
    hb = h_ref[...]
    hs_ref[...] = jnp.dot(hb, ai_ref[...], preferred_element_type=jnp.float32)
    gs_ref[...] = jnp.dot(hb, aa_ref[...], preferred_element_type=jnp.float32)
    z = jnp.dot(hb, wna_ref[...], preferred_element_type=jnp.float32)
    uv_ref[...] = jnp.dot(z, w2_ref[...], preferred_element_type=jnp.float32)


def _final_body(h_ref, sint_ref, cnt_ref, sa_ref, den_ref, wid_ref, wad_ref,
                ci_ref, ca_ref, wu1_ref, wu2_ref, bu_ref, out_ref):
    hb = h_ref[...]
    si = sint_ref[0] + sint_ref[1]
    cnt = cnt_ref[0] + cnt_ref[1]
    sa = sa_ref[0] + sa_ref[1]
    den = den_ref[0] + den_ref[1]
    pi = jnp.dot(hb, wid_ref[...], preferred_element_type=jnp.float32) + ci_ref[...]
    pa = jnp.dot(hb, wad_ref[...], preferred_element_type=jnp.float32) + ca_ref[...]
    msg_i = (si + cnt * pi) / jnp.maximum(cnt, 1.0)
    msg_a = (sa + den * pa) / jnp.maximum(den, 1e-9)
    o = (jnp.dot(msg_i, wu1_ref[...], preferred_element_type=jnp.float32)
         + jnp.dot(msg_a, wu2_ref[...], preferred_element_type=jnp.float32)
         + bu_ref[...])
    out_ref[...] = jnp.maximum(o, 0.0)


def _row_spec(width):
    return pl.BlockSpec((BN, width), lambda i: (i, 0))


def _full_spec(shape):
    nd = len(shape)
    return pl.BlockSpec(shape, lambda i, _n=nd: (0,) * _n)


@jax.jit
def kernel(h, edge_index_intra, edge_index_inter, W_msg_intra, b_msg_intra,
           W_msg_inter, b_msg_inter, ef_intra, ef_inter, W_node_attn, W_attn,
           W_update, b_update):
    f32 = jnp.float32
    h = h.astype(f32)

    ai = W_msg_intra[:, :D].T
    aa = W_msg_inter[:, :D].T
    wid_w = W_msg_intra[:, D:2 * D].T
    wad_w = W_msg_inter[:, D:2 * D].T
    ci = (W_msg_intra[:, 2 * D:] @ ef_intra + b_msg_intra)[None, :]
    ca = (W_msg_inter[:, 2 * D:] @ ef_inter + b_msg_inter)[None, :]
    wna_t = W_node_attn.T
    w2 = W_attn.reshape(2, ATTN).T
    wu1 = W_update[:, :D].T
    wu2 = W_update[:, D:].T
    bu = b_update[None, :]

    hs, gs, uv = pl.pallas_call(
        _prep_body,
        grid=(N // BN,),
        in_specs=[
            _row_spec(D),
            _full_spec((D, D)),
            _full_spec((D, D)),
            _full_spec((D, ATTN)),
            _full_spec((ATTN, 2)),
        ],
        out_specs=[_row_spec(D), _row_spec(D), _row_spec(2)],
        out_shape=[
            jax.ShapeDtypeStruct((N, D), f32),
            jax.ShapeDtypeStruct((N, D), f32),
            jax.ShapeDtypeStruct((N, 2), f32),
        ],
    )(h, ai, aa, wna_t, w2)

    u = uv[:, 0] + 0.0
    v = uv[:, 1] + 0.0

    src_i = edge_index_intra[0].astype(jnp.int32).reshape(NW, NGRP, GRP, CH)
    dst_i = edge_index_intra[1].astype(jnp.int32).reshape(NW, NGRP, GRP, CH)
    src_a = edge_index_inter[0].astype(jnp.int32).reshape(NW, NGRP, GRP, CH)
    dst_a = edge_index_inter[1].astype(jnp.int32).reshape(NW, NGRP, GRP, CH)

    sint, cnt, sa, den = _sc_edges(src_i, dst_i, src_a, dst_a, hs, gs, u, v)

    out = pl.pallas_call(
        _final_body,
        grid=(N // BN,),
        in_specs=[
            _row_spec(D),
            pl.BlockSpec((NC, BN, D), lambda i: (0, i, 0)),
            pl.BlockSpec((NC, BN, 1), lambda i: (0, i, 0)),
            pl.BlockSpec((NC, BN, D), lambda i: (0, i, 0)),
            pl.BlockSpec((NC, BN, 1), lambda i: (0, i, 0)),
            _full_spec((D, D)),
            _full_spec((D, D)),
            _full_spec((1, D)),
            _full_spec((1, D)),
            _full_spec((D, D)),
            _full_spec((D, D)),
            _full_spec((1, D)),
        ],
        out_specs=_row_spec(D),
        out_shape=jax.ShapeDtypeStruct((N, D), f32),
    )(h, sint, cnt[:, :N, None], sa, den[:, :N, None], wid_w, wad_w, ci, ca,
      wu1, wu2, bu)

    return out


# speedup vs baseline: 1.3040x; 1.0009x over previous
import jax
import jax.numpy as jnp
from jax import lax
from jax.experimental import pallas as pl
from jax.experimental.pallas import tpu as pltpu
from jax.experimental.pallas import tpu_sc as plsc

N = 10000
E = 320000
D = 128
ATTN = 64

NC = 2
NS = 16
NW = NC * NS
EPT = E // NW
CH = 80
NCHT = EPT // CH
GRP = 25
NGRP = NCHT // GRP
NB = 3
ROW_A = 640
ROW_B = N - 15 * ROW_A
VEC_A = 1280
NVP = 8 * VEC_A
ZR = 80


def _sc_body(src_i, dst_i, src_a, dst_a, hs_hbm, gs_hbm, u_hbm, v_hbm,
             sint_out, cnt_out, sa_out, den_out,
             src_g, dst_g, ub0, ub1, ub2, vb0, vb1, vb2, ex0, ex1, ex2,
             ones, zvec, rows0, rows1, rows2,
             rsem0, rsem1, rsem2, usem0, usem1, usem2,
             vsem0, vsem1, vsem2, srsem0, srsem1, srsem2,
             svsem0, svsem1, svsem2,
             sh_rows, sh_vec):
    c = lax.axis_index("c")
    s = lax.axis_index("s")
    wid = c * NS + s
    f32 = jnp.float32
    zero16 = jnp.zeros((16,), f32)
    rows = (rows0, rows1, rows2)
    ub = (ub0, ub1, ub2)
    vb = (vb0, vb1, vb2)
    ex = (ex0, ex1, ex2)
    rsem = (rsem0, rsem1, rsem2)
    usem = (usem0, usem1, usem2)
    vsem = (vsem0, vsem1, vsem2)
    srsem = (srsem0, srsem1, srsem2)
    svsem = (svsem0, svsem1, svsem2)

    @pl.loop(0, VEC_A // 16)
    def _(i):
        zvec[pl.ds(i * 16, 16)] = zero16

    for j in range(CH // 16):
        ones[pl.ds(j * 16, 16)] = jnp.ones((16,), f32)

    def zero_shared():
        @pl.loop(0, ZR)
        def _(r):
            for j in range(D // 16):
                rows0[r, pl.ds(j * 16, 16)] = zero16

        @pl.when(s < 15)
        def _():
            for b in range(ROW_A // ZR):
                pltpu.async_copy(rows0,
                                 sh_rows.at[pl.ds(s * ROW_A + b * ZR, ZR)],
                                 rsem0)
            for b in range(ROW_A // ZR):
                pltpu.make_async_copy(
                    rows0, sh_rows.at[pl.ds(s * ROW_A + b * ZR, ZR)],
                    rsem0).wait()

        @pl.when(s == 15)
        def _():
            for b in range(ROW_B // ZR):
                pltpu.async_copy(rows0,
                                 sh_rows.at[pl.ds(15 * ROW_A + b * ZR, ZR)],
                                 rsem0)
            for b in range(ROW_B // ZR):
                pltpu.make_async_copy(
                    rows0, sh_rows.at[pl.ds(15 * ROW_A + b * ZR, ZR)],
                    rsem0).wait()

        @pl.when(s < 8)
        def _():
            pltpu.sync_copy(zvec, sh_vec.at[pl.ds(s * VEC_A, VEC_A)])

    def flush(out3, outv):
        @pl.when(s < 15)
        def _():
            pltpu.sync_copy(sh_rows.at[pl.ds(s * ROW_A, ROW_A)],
                            out3.at[c, pl.ds(s * ROW_A, ROW_A)])

        @pl.when(s == 15)
        def _():
            pltpu.sync_copy(sh_rows.at[pl.ds(15 * ROW_A, ROW_B)],
                            out3.at[c, pl.ds(15 * ROW_A, ROW_B)])

        @pl.when(s < 8)
        def _():
            pltpu.sync_copy(sh_vec.at[pl.ds(s * VEC_A, VEC_A)],
                            outv.at[c, pl.ds(s * VEC_A, VEC_A)])

    def gather_start(kk, b, table, attn):
        pltpu.async_copy(table.at[src_g.at[kk]], rows[b], rsem[b])
        if attn:
            pltpu.async_copy(u_hbm.at[src_g.at[kk]], ub[b], usem[b])
            pltpu.async_copy(v_hbm.at[dst_g.at[kk]], vb[b], vsem[b])

    def gather_wait(kk, b, table, attn):
        pltpu.make_async_copy(table.at[src_g.at[kk]], rows[b], rsem[b]).wait()
        if attn:
            pltpu.make_async_copy(u_hbm.at[src_g.at[kk]], ub[b], usem[b]).wait()
            pltpu.make_async_copy(v_hbm.at[dst_g.at[kk]], vb[b], vsem[b]).wait()

    def scatter_start(kk, b, attn):
        pltpu.async_copy(rows[b], sh_rows.at[dst_g.at[kk]], srsem[b], add=True)
        vsrc = ex[b] if attn else ones
        pltpu.async_copy(vsrc, sh_vec.at[dst_g.at[kk]], svsem[b], add=True)

    def scatter_wait(kk, b, attn):
        pltpu.make_async_copy(rows[b], sh_rows.at[dst_g.at[kk]], srsem[b]).wait()
        vsrc = ex[b] if attn else ones
        pltpu.make_async_copy(vsrc, sh_vec.at[dst_g.at[kk]], svsem[b]).wait()

    def compute_scale(kk, b):
        for j in range(CH // 16):
            sl = pl.ds(j * 16, 16)
            a = ub[b][sl] + vb[b][sl]
            a = jnp.maximum(a, a * 0.01)
            ex[b][sl] = jnp.exp(a)

        @pl.loop(0, CH // 16)
        def _(g):
            r0 = g * 16
            exv = ex[b][pl.ds(r0, 16)]
            for l in range(16):
                ev = jnp.broadcast_to(exv[l], (16,))
                for j in range(D // 16):
                    sl = pl.ds(j * 16, 16)
                    rows[b][r0 + l, sl] = rows[b][r0 + l, sl] * ev

    def chunk(kk, b, table, attn):
        gather_wait(kk, b, table, attn)
        bp = (b + NB - 1) % NB
        kkm1 = jnp.maximum(kk - 1, 0)

        @pl.when(kk >= 1)
        def _():
            scatter_wait(kkm1, bp, attn)

        @pl.when(kk + NB - 1 < GRP)
        def _():
            gather_start(jnp.minimum(kk + NB - 1, GRP - 1), bp, table, attn)

        if attn:
            compute_scale(kk, b)
        scatter_start(kk, b, attn)

    def phase(table, si3, di3, attn, out3, outv):
        zero_shared()
        plsc.subcore_barrier()

        @pl.loop(0, NGRP)
        def _(g):
            pltpu.sync_copy(si3.at[wid, g], src_g)
            pltpu.sync_copy(di3.at[wid, g], dst_g)
            for i in range(NB - 1):
                gather_start(i, i, table, attn)

            @pl.loop(0, GRP - 1, step=NB)
            def _(k):
                for i in range(NB):
                    chunk(k + i, i, table, attn)

            chunk(GRP - 1, (GRP - 1) % NB, table, attn)
            scatter_wait(GRP - 1, (GRP - 1) % NB, attn)

        plsc.subcore_barrier()
        flush(out3, outv)

    phase(hs_hbm, src_i, dst_i, False, sint_out, cnt_out)
    phase(gs_hbm, src_a, dst_a, True, sa_out, den_out)


_sc_edges = pl.kernel(
    _sc_body,
    out_type=(
        jax.ShapeDtypeStruct((NC, N, D), jnp.float32),
        jax.ShapeDtypeStruct((NC, NVP), jnp.float32),
        jax.ShapeDtypeStruct((NC, N, D), jnp.float32),
        jax.ShapeDtypeStruct((NC, NVP), jnp.float32),
    ),
    mesh=plsc.VectorSubcoreMesh(core_axis_name="c", subcore_axis_name="s"),
    scratch_types=(
        [pltpu.VMEM((GRP, CH), jnp.int32)] * 2
        + [pltpu.VMEM((CH,), jnp.float32)] * 10
        + [pltpu.VMEM((VEC_A,), jnp.float32)]
        + [pltpu.VMEM((CH, D), jnp.float32)] * 3
        + [pltpu.SemaphoreType.DMA] * 15
        + [pltpu.VMEM_SHARED((N, D), jnp.float32),
           pltpu.VMEM_SHARED((NVP,), jnp.float32)]
    ),
)


BN = 2000


def _prep_body(h_ref, ai_ref, aa_ref, wna_ref, w2_ref, hs_ref, gs_ref, uv_ref):
    hb = h_ref[...]
    hs_ref[...] = jnp.dot(hb, ai_ref[...], preferred_element_type=jnp.float32)
    gs_ref[...] = jnp.dot(hb, aa_ref[...], preferred_element_type=jnp.float32)
    z = jnp.dot(hb, wna_ref[...], preferred_element_type=jnp.float32)
    uv_ref[...] = jnp.dot(z, w2_ref[...], preferred_element_type=jnp.float32)


def _final_body(h_ref, sint_ref, cnt_ref, sa_ref, den_ref, wid_ref, wad_ref,
                ci_ref, ca_ref, wu1_ref, wu2_ref, bu_ref, out_ref):
    hb = h_ref[...]
    si = sint_ref[0] + sint_ref[1]
    cnt = cnt_ref[0] + cnt_ref[1]
    sa = sa_ref[0] + sa_ref[1]
    den = den_ref[0] + den_ref[1]
    pi = jnp.dot(hb, wid_ref[...], preferred_element_type=jnp.float32) + ci_ref[...]
    pa = jnp.dot(hb, wad_ref[...], preferred_element_type=jnp.float32) + ca_ref[...]
    msg_i = (si + cnt * pi) / jnp.maximum(cnt, 1.0)
    msg_a = (sa + den * pa) / jnp.maximum(den, 1e-9)
    o = (jnp.dot(msg_i, wu1_ref[...], preferred_element_type=jnp.float32)
         + jnp.dot(msg_a, wu2_ref[...], preferred_element_type=jnp.float32)
         + bu_ref[...])
    out_ref[...] = jnp.maximum(o, 0.0)


def _row_spec(width):
    return pl.BlockSpec((BN, width), lambda i: (i, 0))


def _full_spec(shape):
    nd = len(shape)
    return pl.BlockSpec(shape, lambda i, _n=nd: (0,) * _n)


@jax.jit
def kernel(h, edge_index_intra, edge_index_inter, W_msg_intra, b_msg_intra,
           W_msg_inter, b_msg_inter, ef_intra, ef_inter, W_node_attn, W_attn,
           W_update, b_update):
    f32 = jnp.float32
    h = h.astype(f32)

    ai = W_msg_intra[:, :D].T
    aa = W_msg_inter[:, :D].T
    wid_w = W_msg_intra[:, D:2 * D].T
    wad_w = W_msg_inter[:, D:2 * D].T
    ci = (W_msg_intra[:, 2 * D:] @ ef_intra + b_msg_intra)[None, :]
    ca = (W_msg_inter[:, 2 * D:] @ ef_inter + b_msg_inter)[None, :]
    wna_t = W_node_attn.T
    w2 = W_attn.reshape(2, ATTN).T
    wu1 = W_update[:, :D].T
    wu2 = W_update[:, D:].T
    bu = b_update[None, :]

    hs, gs, uv = pl.pallas_call(
        _prep_body,
        grid=(N // BN,),
        in_specs=[
            _row_spec(D),
            _full_spec((D, D)),
            _full_spec((D, D)),
            _full_spec((D, ATTN)),
            _full_spec((ATTN, 2)),
        ],
        out_specs=[_row_spec(D), _row_spec(D), _row_spec(2)],
        out_shape=[
            jax.ShapeDtypeStruct((N, D), f32),
            jax.ShapeDtypeStruct((N, D), f32),
            jax.ShapeDtypeStruct((N, 2), f32),
        ],
    )(h, ai, aa, wna_t, w2)

    u = uv[:, 0] + 0.0
    v = uv[:, 1] + 0.0

    src_i = edge_index_intra[0].astype(jnp.int32).reshape(NW, NGRP, GRP, CH)
    dst_i = edge_index_intra[1].astype(jnp.int32).reshape(NW, NGRP, GRP, CH)
    src_a = edge_index_inter[0].astype(jnp.int32).reshape(NW, NGRP, GRP, CH)
    dst_a = edge_index_inter[1].astype(jnp.int32).reshape(NW, NGRP, GRP, CH)

    sint, cnt, sa, den = _sc_edges(src_i, dst_i, src_a, dst_a, hs, gs, u, v)

    out = pl.pallas_call(
        _final_body,
        grid=(N // BN,),
        in_specs=[
            _row_spec(D),
            pl.BlockSpec((NC, BN, D), lambda i: (0, i, 0)),
            pl.BlockSpec((NC, BN, 1), lambda i: (0, i, 0)),
            pl.BlockSpec((NC, BN, D), lambda i: (0, i, 0)),
            pl.BlockSpec((NC, BN, 1), lambda i: (0, i, 0)),
            _full_spec((D, D)),
            _full_spec((D, D)),
            _full_spec((1, D)),
            _full_spec((1, D)),
            _full_spec((D, D)),
            _full_spec((D, D)),
            _full_spec((1, D)),
        ],
        out_specs=_row_spec(D),
        out_shape=jax.ShapeDtypeStruct((N, D), f32),
    )(h, sint, cnt[:, :N, None], sa, den[:, :N, None], wid_w, wad_w, ci, ca,
      wu1, wu2, bu)

    return out
